# baseline (speedup 1.0000x reference)
import functools

import jax
import jax.numpy as jnp
from jax import lax
from jax.experimental import pallas as pl
from jax.experimental.pallas import tpu as pltpu
from jax.experimental.pallas import tpu_sc as plsc

BATCH = 16384
DIM = 64
WPR = DIM // 2
PACK = 4
PDIM = WPR * PACK
NUM_CORES = 2
NUM_SUBCORES = 16
NUM_WORKERS = NUM_CORES * NUM_SUBCORES
BPW = BATCH // NUM_WORKERS
LANES = 16
CH = 256
NCH = BPW // CH


def _pack_bf16_words(table):
    bits = lax.bitcast_convert_type(table, jnp.int32)
    rnd = bits + jnp.int32(0x8000)
    lo = jnp.bitwise_and(
        lax.shift_right_logical(rnd[:, 0::2], 16), jnp.int32(0xFFFF))
    hi = jnp.bitwise_and(rnd[:, 1::2], jnp.int32(-65536))
    return jnp.bitwise_or(lo, hi).reshape(-1, PDIM)


def _sqrt16(x):
    i = lax.bitcast_convert_type(x, jnp.int32)
    i = jnp.int32(0x5F3759DF) - lax.shift_right_arithmetic(i, jnp.int32(1))
    r = lax.bitcast_convert_type(i, jnp.float32)
    half = x * jnp.float32(0.5)
    for _ in range(4):
        r = r * (jnp.float32(1.5) - half * r * r)
    return x * r


def _expand2(words):
    even = lax.bitcast_convert_type(
        lax.shift_left(words, jnp.int32(16)), jnp.float32)
    odd = lax.bitcast_convert_type(
        jnp.bitwise_and(words, jnp.int32(-65536)), jnp.float32)
    return even, odd


def _transe_body(ent_hbm, rel_hbm, h_hbm, r_hbm, t_hbm, out_hbm,
                 hfull, rfull, tfull, hq, rq, tq,
                 hrows, rrows, trows, out_v,
                 sem_h, sem_r, sem_t):
    wid = lax.axis_index("s") * NUM_CORES + lax.axis_index("c")
    base = wid * BPW

    pltpu.sync_copy(h_hbm.at[pl.ds(base, BPW)], hfull)
    pltpu.sync_copy(r_hbm.at[pl.ds(base, BPW)], rfull)
    pltpu.sync_copy(t_hbm.at[pl.ds(base, BPW)], tfull)

    def idx_body(k, carry):
        off = k * LANES
        two = jnp.int32(2)
        hq[pl.ds(off, LANES)] = lax.shift_right_logical(
            hfull[pl.ds(off, LANES)], two)
        rq[pl.ds(off, LANES)] = lax.shift_right_logical(
            rfull[pl.ds(off, LANES)], two)
        tq[pl.ds(off, LANES)] = lax.shift_right_logical(
            tfull[pl.ds(off, LANES)], two)
        return carry

    lax.fori_loop(0, BPW // LANES, idx_body, jnp.int32(0))

    lanes = lax.iota(jnp.int32, LANES)
    perms = [lanes ^ sh for sh in (8, 4, 2, 1)]

    def chunk_body(k, carry):
        coff = k * CH
        ch_ = pltpu.async_copy(
            ent_hbm.at[hq.at[pl.ds(coff, CH)]], hrows, sem_h)
        cr_ = pltpu.async_copy(
            rel_hbm.at[rq.at[pl.ds(coff, CH)]], rrows, sem_r)
        ct_ = pltpu.async_copy(
            ent_hbm.at[tq.at[pl.ds(coff, CH)]], trows, sem_t)
        ch_.wait()
        cr_.wait()
        ct_.wait()

        def group_body(g, carry2):
            rbase = g * LANES
            three = jnp.int32(3)
            hpar = jnp.bitwise_and(hfull[pl.ds(coff + rbase, LANES)], three)
            rpar = jnp.bitwise_and(rfull[pl.ds(coff + rbase, LANES)], three)
            tpar = jnp.bitwise_and(tfull[pl.ds(coff + rbase, LANES)], three)
            vec = jnp.zeros((LANES,), jnp.float32)
            for j in range(LANES):
                i = rbase + j
                hoff = hpar[j] * WPR
                roff = rpar[j] * WPR
                toff = tpar[j] * WPR
                acc = jnp.zeros((LANES,), jnp.float32)
                for c in range(WPR // LANES):
                    hw = hrows[i, pl.dslice(hoff + c * LANES, LANES)]
                    rw = rrows[i, pl.dslice(roff + c * LANES, LANES)]
                    tw = trows[i, pl.dslice(toff + c * LANES, LANES)]
                    h0, h1 = _expand2(hw)
                    r0, r1 = _expand2(rw)
                    t0, t1 = _expand2(tw)
                    d0 = (h0 - t0) + r0
                    d1 = (h1 - t1) + r1
                    acc = acc + d0 * d0
                    acc = acc + d1 * d1
                for p in perms:
                    acc = acc + acc.at[p].get(mode="promise_in_bounds")
                vec = jnp.where(lanes == j, acc, vec)
            out_v[pl.ds(coff + rbase, LANES)] = _sqrt16(vec)
            return carry2

        lax.fori_loop(0, CH // LANES, group_body, jnp.int32(0))
        return carry

    lax.fori_loop(0, NCH, chunk_body, jnp.int32(0))

    pltpu.sync_copy(out_v, out_hbm.at[pl.ds(base, BPW)])


@jax.jit
def kernel(entity_emb, relation_emb, h, r, t):
    ent_w = _pack_bf16_words(entity_emb)
    rel_w = _pack_bf16_words(relation_emb)
    mesh = plsc.VectorSubcoreMesh(core_axis_name="c", subcore_axis_name="s")
    f = functools.partial(
        pl.kernel,
        mesh=mesh,
        out_type=jax.ShapeDtypeStruct((BATCH,), jnp.float32),
        scratch_types=[
            pltpu.VMEM((BPW,), jnp.int32),
            pltpu.VMEM((BPW,), jnp.int32),
            pltpu.VMEM((BPW,), jnp.int32),
            pltpu.VMEM((BPW,), jnp.int32),
            pltpu.VMEM((BPW,), jnp.int32),
            pltpu.VMEM((BPW,), jnp.int32),
            pltpu.VMEM((CH, PDIM), jnp.int32),
            pltpu.VMEM((CH, PDIM), jnp.int32),
            pltpu.VMEM((CH, PDIM), jnp.int32),
            pltpu.VMEM((BPW,), jnp.float32),
            pltpu.SemaphoreType.DMA,
            pltpu.SemaphoreType.DMA,
            pltpu.SemaphoreType.DMA,
        ],
    )(_transe_body)
    return f(ent_w, rel_w, h, r.astype(jnp.int32), t)


# speedup vs baseline: 3.6207x; 3.6207x over previous
import functools

import jax
import jax.numpy as jnp
from jax import lax
from jax.experimental import pallas as pl
from jax.experimental.pallas import tpu as pltpu
from jax.experimental.pallas import tpu_sc as plsc

BATCH = 16384
DIM = 64
PDIM = 128
NUM_CORES = 2
NUM_SUBCORES = 16
NUM_WORKERS = NUM_CORES * NUM_SUBCORES
BPW = BATCH // NUM_WORKERS
LANES = 16
CH = 256
NCH = BPW // CH


def _sqrt16(x):
    i = lax.bitcast_convert_type(x, jnp.int32)
    i = jnp.int32(0x5F3759DF) - lax.shift_right_arithmetic(i, jnp.int32(1))
    r = lax.bitcast_convert_type(i, jnp.float32)
    half = x * jnp.float32(0.5)
    for _ in range(4):
        r = r * (jnp.float32(1.5) - half * r * r)
    return x * r


def _transe_body(ent_hbm, rel_hbm, h_hbm, r_hbm, t_hbm, out_hbm,
                 hidx_v, ridx_v, tidx_v, hrows, rrows, trows, out_v,
                 sem_h, sem_r, sem_t):
    wid = lax.axis_index("s") * NUM_CORES + lax.axis_index("c")
    base = wid * BPW

    pltpu.sync_copy(h_hbm.at[pl.ds(base, BPW)], hidx_v)
    pltpu.sync_copy(r_hbm.at[pl.ds(base, BPW)], ridx_v)
    pltpu.sync_copy(t_hbm.at[pl.ds(base, BPW)], tidx_v)

    lanes = lax.iota(jnp.int32, LANES)
    perms = [lanes ^ sh for sh in (8, 4, 2, 1)]

    def chunk_body(k, carry):
        coff = k * CH
        ch_ = pltpu.async_copy(
            ent_hbm.at[hidx_v.at[pl.ds(coff, CH)]], hrows, sem_h)
        cr_ = pltpu.async_copy(
            rel_hbm.at[ridx_v.at[pl.ds(coff, CH)]], rrows, sem_r)
        ct_ = pltpu.async_copy(
            ent_hbm.at[tidx_v.at[pl.ds(coff, CH)]], trows, sem_t)
        ch_.wait()
        cr_.wait()
        ct_.wait()

        def group_body(g, carry2):
            rbase = g * LANES
            vec = jnp.zeros((LANES,), jnp.float32)
            for j in range(LANES):
                i = rbase + j
                acc = jnp.zeros((LANES,), jnp.float32)
                for c in range(DIM // LANES):
                    hv = hrows[i, pl.ds(c * LANES, LANES)]
                    rv = rrows[i, pl.ds(c * LANES, LANES)]
                    tv = trows[i, pl.ds(c * LANES, LANES)]
                    d = (hv - tv) + rv
                    acc = acc + d * d
                for p in perms:
                    acc = acc + acc.at[p].get(mode="promise_in_bounds")
                vec = jnp.where(lanes == j, acc, vec)
            out_v[pl.ds(coff + rbase, LANES)] = _sqrt16(vec)
            return carry2

        lax.fori_loop(0, CH // LANES, group_body, jnp.int32(0))
        return carry

    lax.fori_loop(0, NCH, chunk_body, jnp.int32(0))

    pltpu.sync_copy(out_v, out_hbm.at[pl.ds(base, BPW)])


@jax.jit
def kernel(entity_emb, relation_emb, h, r, t):
    ent_p = jnp.pad(entity_emb, ((0, 0), (0, PDIM - DIM)))
    rel_p = jnp.pad(relation_emb, ((0, 0), (0, PDIM - DIM)))
    mesh = plsc.VectorSubcoreMesh(core_axis_name="c", subcore_axis_name="s")
    f = functools.partial(
        pl.kernel,
        mesh=mesh,
        out_type=jax.ShapeDtypeStruct((BATCH,), jnp.float32),
        scratch_types=[
            pltpu.VMEM((BPW,), jnp.int32),
            pltpu.VMEM((BPW,), jnp.int32),
            pltpu.VMEM((BPW,), jnp.int32),
            pltpu.VMEM((CH, PDIM), jnp.float32),
            pltpu.VMEM((CH, PDIM), jnp.float32),
            pltpu.VMEM((CH, PDIM), jnp.float32),
            pltpu.VMEM((BPW,), jnp.float32),
            pltpu.SemaphoreType.DMA,
            pltpu.SemaphoreType.DMA,
            pltpu.SemaphoreType.DMA,
        ],
    )(_transe_body)
    return f(ent_p, rel_p, h, r.astype(jnp.int32), t)


# speedup vs baseline: 5.2548x; 1.4513x over previous
import functools

import jax
import jax.numpy as jnp
from jax import lax
from jax.experimental import pallas as pl
from jax.experimental.pallas import tpu as pltpu
from jax.experimental.pallas import tpu_sc as plsc

BATCH = 16384
DIM = 64
NUM_CORES = 2
NUM_SUBCORES = 16
NUM_WORKERS = NUM_CORES * NUM_SUBCORES
BPW = BATCH // NUM_WORKERS
LANES = 16
CH = 16
NCH = BPW // CH


def _sqrt16(x):
    i = lax.bitcast_convert_type(x, jnp.int32)
    i = jnp.int32(0x5F3759DF) - lax.shift_right_arithmetic(i, jnp.int32(1))
    r = lax.bitcast_convert_type(i, jnp.float32)
    half = x * jnp.float32(0.5)
    for _ in range(4):
        r = r * (jnp.float32(1.5) - half * r * r)
    return x * r


def _transe_body(ent_hbm, rel_hbm, h_hbm, r_hbm, t_hbm, out_hbm,
                 hfull, rfull, tfull, hbuf, rbuf, tbuf, out_v, sem):
    wid = lax.axis_index("s") * NUM_CORES + lax.axis_index("c")
    base = wid * BPW

    pltpu.sync_copy(h_hbm.at[pl.ds(base, BPW)], hfull)
    pltpu.sync_copy(r_hbm.at[pl.ds(base, BPW)], rfull)
    pltpu.sync_copy(t_hbm.at[pl.ds(base, BPW)], tfull)

    lanes = lax.iota(jnp.int32, LANES)
    perms = [lanes ^ sh for sh in (8, 4, 2, 1)]

    def chunk_body(k, carry):
        off = k * CH
        hidx = hfull[pl.ds(off, CH)]
        ridx = rfull[pl.ds(off, CH)]
        tidx = tfull[pl.ds(off, CH)]
        copies = []
        for j in range(CH):
            copies.append(
                pltpu.async_copy(ent_hbm.at[hidx[j]], hbuf.at[j], sem))
            copies.append(
                pltpu.async_copy(rel_hbm.at[ridx[j]], rbuf.at[j], sem))
            copies.append(
                pltpu.async_copy(ent_hbm.at[tidx[j]], tbuf.at[j], sem))
        for c in copies:
            c.wait()

        vec = jnp.zeros((LANES,), jnp.float32)
        for j in range(CH):
            acc = jnp.zeros((LANES,), jnp.float32)
            for c in range(DIM // LANES):
                hv = hbuf[j, pl.ds(c * LANES, LANES)]
                rv = rbuf[j, pl.ds(c * LANES, LANES)]
                tv = tbuf[j, pl.ds(c * LANES, LANES)]
                d = (hv - tv) + rv
                acc = acc + d * d
            for p in perms:
                acc = acc + acc.at[p].get(mode="promise_in_bounds")
            vec = jnp.where(lanes == j, acc, vec)
        out_v[pl.ds(off, LANES)] = _sqrt16(vec)
        return carry

    lax.fori_loop(0, NCH, chunk_body, jnp.int32(0))

    pltpu.sync_copy(out_v, out_hbm.at[pl.ds(base, BPW)])


@jax.jit
def kernel(entity_emb, relation_emb, h, r, t):
    mesh = plsc.VectorSubcoreMesh(core_axis_name="c", subcore_axis_name="s")
    f = functools.partial(
        pl.kernel,
        mesh=mesh,
        out_type=jax.ShapeDtypeStruct((BATCH,), jnp.float32),
        scratch_types=[
            pltpu.VMEM((BPW,), jnp.int32),
            pltpu.VMEM((BPW,), jnp.int32),
            pltpu.VMEM((BPW,), jnp.int32),
            pltpu.VMEM((CH, DIM), jnp.float32),
            pltpu.VMEM((CH, DIM), jnp.float32),
            pltpu.VMEM((CH, DIM), jnp.float32),
            pltpu.VMEM((BPW,), jnp.float32),
            pltpu.SemaphoreType.DMA,
        ],
    )(_transe_body)
    return f(entity_emb, relation_emb, h, r.astype(jnp.int32), t)


# speedup vs baseline: 5.3748x; 1.0228x over previous
import functools

import jax
import jax.numpy as jnp
from jax import lax
from jax.experimental import pallas as pl
from jax.experimental.pallas import tpu as pltpu
from jax.experimental.pallas import tpu_sc as plsc

BATCH = 16384
DIM = 64
NUM_CORES = 2
NUM_SUBCORES = 16
NUM_WORKERS = NUM_CORES * NUM_SUBCORES
BPW = BATCH // NUM_WORKERS
LANES = 16
CH = 32
NCH = BPW // CH


def _sqrt16(x):
    i = lax.bitcast_convert_type(x, jnp.int32)
    i = jnp.int32(0x5F3759DF) - lax.shift_right_arithmetic(i, jnp.int32(1))
    r = lax.bitcast_convert_type(i, jnp.float32)
    half = x * jnp.float32(0.5)
    for _ in range(4):
        r = r * (jnp.float32(1.5) - half * r * r)
    return x * r


def _transe_body(ent_hbm, rel_hbm, h_hbm, r_hbm, t_hbm, out_hbm,
                 hfull, rfull, tfull, hbuf, rbuf, tbuf, out_v, sem):
    wid = lax.axis_index("s") * NUM_CORES + lax.axis_index("c")
    base = wid * BPW

    pltpu.sync_copy(h_hbm.at[pl.ds(base, BPW)], hfull)
    pltpu.sync_copy(r_hbm.at[pl.ds(base, BPW)], rfull)
    pltpu.sync_copy(t_hbm.at[pl.ds(base, BPW)], tfull)

    lanes = lax.iota(jnp.int32, LANES)
    perms = [lanes ^ sh for sh in (8, 4, 2, 1)]

    def chunk_body(k, carry):
        off = k * CH
        hidx_lo = hfull[pl.ds(off, LANES)]
        hidx_hi = hfull[pl.ds(off + LANES, LANES)]
        ridx_lo = rfull[pl.ds(off, LANES)]
        ridx_hi = rfull[pl.ds(off + LANES, LANES)]
        tidx_lo = tfull[pl.ds(off, LANES)]
        tidx_hi = tfull[pl.ds(off + LANES, LANES)]
        for j in range(CH):
            hi = hidx_lo[j] if j < LANES else hidx_hi[j - LANES]
            ri = ridx_lo[j] if j < LANES else ridx_hi[j - LANES]
            ti = tidx_lo[j] if j < LANES else tidx_hi[j - LANES]
            pltpu.async_copy(ent_hbm.at[hi], hbuf.at[j], sem)
            pltpu.async_copy(rel_hbm.at[ri], rbuf.at[j], sem)
            pltpu.async_copy(ent_hbm.at[ti], tbuf.at[j], sem)
        pltpu.make_async_copy(ent_hbm.at[pl.ds(0, CH)], hbuf, sem).wait()
        pltpu.make_async_copy(rel_hbm.at[pl.ds(0, CH)], rbuf, sem).wait()
        pltpu.make_async_copy(ent_hbm.at[pl.ds(0, CH)], tbuf, sem).wait()

        for g in range(CH // LANES):
            vec = jnp.zeros((LANES,), jnp.float32)
            for jj in range(LANES):
                j = g * LANES + jj
                acc = jnp.zeros((LANES,), jnp.float32)
                for c in range(DIM // LANES):
                    hv = hbuf[j, pl.ds(c * LANES, LANES)]
                    rv = rbuf[j, pl.ds(c * LANES, LANES)]
                    tv = tbuf[j, pl.ds(c * LANES, LANES)]
                    d = (hv - tv) + rv
                    acc = acc + d * d
                for p in perms:
                    acc = acc + acc.at[p].get(mode="promise_in_bounds")
                vec = jnp.where(lanes == jj, acc, vec)
            out_v[pl.ds(off + g * LANES, LANES)] = _sqrt16(vec)
        return carry

    lax.fori_loop(0, NCH, chunk_body, jnp.int32(0))

    pltpu.sync_copy(out_v, out_hbm.at[pl.ds(base, BPW)])


@jax.jit
def kernel(entity_emb, relation_emb, h, r, t):
    mesh = plsc.VectorSubcoreMesh(core_axis_name="c", subcore_axis_name="s")
    f = functools.partial(
        pl.kernel,
        mesh=mesh,
        out_type=jax.ShapeDtypeStruct((BATCH,), jnp.float32),
        scratch_types=[
            pltpu.VMEM((BPW,), jnp.int32),
            pltpu.VMEM((BPW,), jnp.int32),
            pltpu.VMEM((BPW,), jnp.int32),
            pltpu.VMEM((CH, DIM), jnp.float32),
            pltpu.VMEM((CH, DIM), jnp.float32),
            pltpu.VMEM((CH, DIM), jnp.float32),
            pltpu.VMEM((BPW,), jnp.float32),
            pltpu.SemaphoreType.DMA,
        ],
    )(_transe_body)
    return f(entity_emb, relation_emb, h, r.astype(jnp.int32), t)
